# Initial kernel scaffold; baseline (speedup 1.0000x reference)
import jax
import jax.numpy as jnp
from jax.experimental import pallas as pl


def kernel(user_emb, item_emb, adj_values, adj_indices):
    raise NotImplementedError("write your pallas kernel here")


# speedup vs baseline: 6.3817x; 6.3817x over previous
import functools

import jax
import jax.numpy as jnp
from jax import lax
from jax.experimental import pallas as pl
from jax.experimental.pallas import tpu as pltpu
from jax.experimental.pallas import tpu_sc as plsc

_N_USERS = 50000
_N = 100000
_E = 1600000
_D = 32
_L = 3

_NS = 16
_Q = 8
_RPT = 6256
_NP = _RPT * _NS
_EP = 1638400
_EROWS = _EP // 128
_TROWS = _EROWS // _NS
_WROWS = 16
_NWIN = _TROWS // _WROWS


def _sc_propagate(table0, src2, dst2, val2):
    mesh = plsc.VectorSubcoreMesh(core_axis_name="c", subcore_axis_name="s")
    out_sds = jax.ShapeDtypeStruct((4 * _NP, _Q), jnp.float32)

    @functools.partial(
        pl.kernel,
        out_type=[out_sds] * _L,
        mesh=mesh,
        scratch_types=[
            pltpu.VMEM((_WROWS, 128), jnp.int32),
            pltpu.VMEM((_WROWS, 128), jnp.int32),
            pltpu.VMEM((_WROWS, 128), jnp.float32),
            pltpu.VMEM((_WROWS, 128, _Q), jnp.float32),
            pltpu.VMEM((128, _Q), jnp.float32),
            pltpu.VMEM_SHARED((_NP, _Q), jnp.float32),
            pltpu.SemaphoreType.DMA,
            pltpu.SemaphoreType.DMA,
        ],
        compiler_params=pltpu.CompilerParams(use_tc_tiling_on_sc=False,
                                             needs_layout_passes=False),
    )
    def body(tab_hbm, src_hbm, dst_hbm, val_hbm, o1, o2, o3,
             srcb, dstb, valb, rowsb, zbuf, acc, gsem, ssem):
        c = lax.axis_index("c")
        s = lax.axis_index("s")
        r0 = s * _RPT
        erow0 = s * _TROWS

        iota = lax.broadcasted_iota(jnp.int32, (16,), 0)
        patv = iota >> 3
        colv = iota & 7
        lane_lo = patv == 0

        zeros16 = jnp.zeros((16,), jnp.float32)

        @pl.loop(0, 64)
        def _zfill(k):
            plsc.store_scatter(zbuf, [patv + 2 * k, colv], zeros16)

        tables = [tab_hbm, o1, o2]
        outs = [o1, o2, o3]
        for l in range(_L):
            @pl.loop(0, 2)
            def _pass(qi, l=l):
                q = 2 * c + qi
                qrow = q * _NP

                @pl.loop(0, 48)
                def _zero(i):
                    pltpu.sync_copy(zbuf, acc.at[pl.ds(r0 + i * 128, 128)])
                pltpu.sync_copy(zbuf.at[pl.ds(0, 112)],
                                acc.at[pl.ds(r0 + 48 * 128, 112)])
                plsc.subcore_barrier()

                @pl.loop(0, _NWIN)
                def _window(w):
                    wr = erow0 + w * _WROWS
                    pltpu.sync_copy(src_hbm.at[pl.ds(wr, _WROWS)], srcb)
                    pltpu.sync_copy(dst_hbm.at[pl.ds(wr, _WROWS)], dstb)
                    pltpu.sync_copy(val_hbm.at[pl.ds(wr, _WROWS)], valb)
                    @pl.loop(0, _WROWS)
                    def _shift(j):
                        @pl.loop(0, 128, step=16)
                        def _shift_k(k):
                            srcb[j, pl.ds(k, 16)] = srcb[j, pl.ds(k, 16)] + qrow
                    gathers = [
                        pltpu.async_copy(tables[l].at[srcb.at[j]],
                                         rowsb.at[j], gsem)
                        for j in range(_WROWS)
                    ]
                    for g in gathers:
                        g.wait()
                    @pl.loop(0, _WROWS)
                    def _scalej(j):
                        jv = jnp.broadcast_to(j, (16,))

                        @pl.loop(0, 8)
                        def _scale(m):
                            vv = valb[j, pl.ds(m * 16, 16)]
                            for t in range(8):
                                e = m * 16 + 2 * t
                                pair = jnp.where(lane_lo, vv[2 * t], vv[2 * t + 1])
                                ridx = patv + e
                                rows = plsc.load_gather(rowsb, [jv, ridx, colv])
                                plsc.store_scatter(rowsb, [jv, ridx, colv],
                                                   rows * pair)
                    scats = [
                        pltpu.async_copy(rowsb.at[j], acc.at[dstb.at[j]], ssem,
                                         add=True)
                        for j in range(_WROWS)
                    ]
                    for sc_ in scats:
                        sc_.wait()

                plsc.subcore_barrier()
                pltpu.sync_copy(acc.at[pl.ds(r0, _RPT)],
                                outs[l].at[pl.ds(qrow + r0, _RPT)])

    return body(table0, src2, dst2, val2)


def _mean3(a, b, c):
    rows = a.shape[0]
    grid = 8
    bs = rows // grid

    def body(x_ref, y_ref, z_ref, o_ref):
        o_ref[...] = (x_ref[...] + y_ref[...] + z_ref[...]) * (1.0 / 3.0)

    spec = pl.BlockSpec((bs, 128), lambda i: (i, 0))
    return pl.pallas_call(
        body,
        out_shape=jax.ShapeDtypeStruct(a.shape, a.dtype),
        grid=(grid,),
        in_specs=[spec, spec, spec],
        out_specs=spec,
    )(a, b, c)


def kernel(user_emb, item_emb, adj_values, adj_indices):
    f32 = jnp.float32
    src = adj_indices[0]
    dst = adj_indices[1]
    ego = jnp.concatenate([user_emb, item_emb], axis=0)
    zpad = jnp.zeros((_NP - _N, _Q), f32)
    table0 = jnp.concatenate(
        [ego[:, 0:8], zpad, ego[:, 8:16], zpad,
         ego[:, 16:24], zpad, ego[:, 24:32], zpad], axis=0)

    pad_n = _EP - _E
    pad_idx = (jnp.arange(pad_n, dtype=jnp.int32) * 61) % _N
    src2 = jnp.concatenate([src, pad_idx]).reshape(_EROWS, 128)
    dst2 = jnp.concatenate([dst, pad_idx]).reshape(_EROWS, 128)
    val2 = jnp.concatenate(
        [adj_values, jnp.zeros((pad_n,), f32)]).reshape(_EROWS, 128)

    o1, o2, o3 = _sc_propagate(table0, src2, dst2, val2)

    m = _mean3(o1.reshape(-1, 128), o2.reshape(-1, 128), o3.reshape(-1, 128))
    m = m.reshape(4, _NP, _Q)
    user_all = jnp.concatenate([m[q, :_N_USERS] for q in range(4)], axis=1)
    item_all = jnp.concatenate([m[q, _N_USERS:_N] for q in range(4)], axis=1)
    return (user_all, item_all)


# speedup vs baseline: 9.2933x; 1.4562x over previous
import functools

import jax
import jax.numpy as jnp
from jax import lax
from jax.experimental import pallas as pl
from jax.experimental.pallas import tpu as pltpu
from jax.experimental.pallas import tpu_sc as plsc

_N_USERS = 50000
_N = 100000
_E = 1600000
_D = 32
_L = 3

_NS = 16
_Q = 8
_RPT = 6256
_NP = _RPT * _NS
_EP = 1638400
_EROWS = _EP // 128
_TROWS = _EROWS // _NS
_WROWS = 16
_NWIN = _TROWS // _WROWS


def _sc_propagate(table0, src2, dst2, val2):
    mesh = plsc.VectorSubcoreMesh(core_axis_name="c", subcore_axis_name="s")
    out_sds = jax.ShapeDtypeStruct((4 * _NP, _Q), jnp.float32)

    @functools.partial(
        pl.kernel,
        out_type=[out_sds] * _L,
        mesh=mesh,
        scratch_types=[
            pltpu.VMEM((_WROWS, 128), jnp.int32),
            pltpu.VMEM((_WROWS, 128), jnp.int32),
            pltpu.VMEM((_WROWS, 128), jnp.float32),
            pltpu.VMEM((_WROWS, 128, _Q), jnp.float32),
            pltpu.VMEM((_WROWS, 128), jnp.int32),
            pltpu.VMEM((_WROWS, 128), jnp.int32),
            pltpu.VMEM((_WROWS, 128), jnp.float32),
            pltpu.VMEM((_WROWS, 128, _Q), jnp.float32),
            pltpu.VMEM((128, _Q), jnp.float32),
            pltpu.VMEM_SHARED((_NP, _Q), jnp.float32),
            pltpu.SemaphoreType.DMA,
            pltpu.SemaphoreType.DMA,
            pltpu.SemaphoreType.DMA,
        ],
        compiler_params=pltpu.CompilerParams(use_tc_tiling_on_sc=False,
                                             needs_layout_passes=False),
    )
    def body(tab_hbm, src_hbm, dst_hbm, val_hbm, o1, o2, o3,
             srcA, dstA, valA, rowsA, srcB, dstB, valB, rowsB,
             zbuf, acc, lsem, gsem, ssem):
        c = lax.axis_index("c")
        s = lax.axis_index("s")
        r0 = s * _RPT
        erow0 = s * _TROWS

        iota = lax.broadcasted_iota(jnp.int32, (16,), 0)
        patv = iota >> 3
        colv = iota & 7
        lane_lo = patv == 0

        zeros16 = jnp.zeros((16,), jnp.float32)

        @pl.loop(0, 64)
        def _zfill(k):
            plsc.store_scatter(zbuf, [patv + 2 * k, colv], zeros16)

        def load_idx(w, sb, db, vb):
            wr = jnp.minimum(erow0 + w * _WROWS, _EROWS - _WROWS)
            pltpu.async_copy(src_hbm.at[pl.ds(wr, _WROWS)], sb, lsem)
            pltpu.async_copy(dst_hbm.at[pl.ds(wr, _WROWS)], db, lsem)
            pltpu.async_copy(val_hbm.at[pl.ds(wr, _WROWS)], vb, lsem)

        def drain_idx(sb, db, vb):
            pltpu.make_async_copy(src_hbm.at[pl.ds(0, _WROWS)], sb, lsem).wait()
            pltpu.make_async_copy(dst_hbm.at[pl.ds(0, _WROWS)], db, lsem).wait()
            pltpu.make_async_copy(val_hbm.at[pl.ds(0, _WROWS)], vb, lsem).wait()

        def shift(sb, qrow):
            @pl.loop(0, _WROWS)
            def _shift(j):
                for k in range(0, 128, 16):
                    sb[j, pl.ds(k, 16)] = sb[j, pl.ds(k, 16)] + qrow

        def fire_gathers(tab, sb, rb):
            for j in range(_WROWS):
                pltpu.async_copy(tab.at[sb.at[j]], rb.at[j], gsem)

        def drain_gathers(tab, sb, rb):
            for j in range(_WROWS):
                pltpu.make_async_copy(tab.at[sb.at[j]], rb.at[j], gsem).wait()

        def fire_scatters(db, rb):
            for j in range(_WROWS):
                pltpu.async_copy(rb.at[j], acc.at[db.at[j]], ssem, add=True)

        def drain_scatters(db, rb):
            for j in range(_WROWS):
                pltpu.make_async_copy(rb.at[j], acc.at[db.at[j]], ssem).wait()

        pats = [patv + 2 * t for t in range(8)]

        def scale(vb, rb):
            @plsc.parallel_loop(0, _WROWS, 1, unroll=2)
            def _scalej(j):
                jv = jnp.broadcast_to(j, (16,))

                @pl.loop(0, 8)
                def _scalem(m):
                    vv = vb[j, pl.ds(m * 16, 16)]
                    basev = jnp.broadcast_to(m * 16, (16,))
                    for t in range(8):
                        pair = lax.gather(
                            vv, pats[t][:, None],
                            lax.GatherDimensionNumbers(
                                offset_dims=(), collapsed_slice_dims=(0,),
                                start_index_map=(0,)),
                            slice_sizes=(1,),
                            mode=lax.GatherScatterMode.PROMISE_IN_BOUNDS)
                        ridx = pats[t] + basev
                        rows = plsc.load_gather(rb, [jv, ridx, colv])
                        plsc.store_scatter(rb, [jv, ridx, colv], rows * pair)

        tables = [tab_hbm, o1, o2]
        outs = [o1, o2, o3]
        for l in range(_L):
            @pl.loop(0, 2)
            def _pass(qi, l=l):
                q = 2 * c + qi
                qrow = q * _NP
                tab = tables[l]

                @pl.loop(0, 48)
                def _zero(i):
                    pltpu.sync_copy(zbuf, acc.at[pl.ds(r0 + i * 128, 128)])
                pltpu.sync_copy(zbuf.at[pl.ds(0, 112)],
                                acc.at[pl.ds(r0 + 48 * 128, 112)])
                plsc.subcore_barrier()

                load_idx(0, srcA, dstA, valA)
                drain_idx(srcA, dstA, valA)
                shift(srcA, qrow)
                fire_gathers(tab, srcA, rowsA)

                @pl.loop(0, _NWIN // 2)
                def _iter(i):
                    @pl.when(i > 0)
                    def _():
                        drain_scatters(dstB, rowsB)
                    load_idx(2 * i + 1, srcB, dstB, valB)
                    drain_idx(srcB, dstB, valB)
                    shift(srcB, qrow)
                    fire_gathers(tab, srcB, rowsB)
                    drain_gathers(tab, srcA, rowsA)
                    scale(valA, rowsA)
                    fire_scatters(dstA, rowsA)

                    drain_scatters(dstA, rowsA)
                    load_idx(2 * i + 2, srcA, dstA, valA)
                    drain_idx(srcA, dstA, valA)
                    shift(srcA, qrow)
                    fire_gathers(tab, srcA, rowsA)
                    drain_gathers(tab, srcB, rowsB)
                    scale(valB, rowsB)
                    fire_scatters(dstB, rowsB)

                drain_scatters(dstB, rowsB)
                drain_gathers(tab, srcA, rowsA)

                plsc.subcore_barrier()
                pltpu.sync_copy(acc.at[pl.ds(r0, _RPT)],
                                outs[l].at[pl.ds(qrow + r0, _RPT)])

    return body(table0, src2, dst2, val2)


def _mean3(a, b, c):
    rows = a.shape[0]
    grid = 8
    bs = rows // grid

    def body(x_ref, y_ref, z_ref, o_ref):
        o_ref[...] = (x_ref[...] + y_ref[...] + z_ref[...]) * (1.0 / 3.0)

    spec = pl.BlockSpec((bs, 128), lambda i: (i, 0))
    return pl.pallas_call(
        body,
        out_shape=jax.ShapeDtypeStruct(a.shape, a.dtype),
        grid=(grid,),
        in_specs=[spec, spec, spec],
        out_specs=spec,
    )(a, b, c)


def kernel(user_emb, item_emb, adj_values, adj_indices):
    f32 = jnp.float32
    src = adj_indices[0]
    dst = adj_indices[1]
    ego = jnp.concatenate([user_emb, item_emb], axis=0)
    zpad = jnp.zeros((_NP - _N, _Q), f32)
    table0 = jnp.concatenate(
        [ego[:, 0:8], zpad, ego[:, 8:16], zpad,
         ego[:, 16:24], zpad, ego[:, 24:32], zpad], axis=0)

    pad_n = _EP - _E
    pad_idx = (jnp.arange(pad_n, dtype=jnp.int32) * 61) % _N
    src2 = jnp.concatenate([src, pad_idx]).reshape(_EROWS, 128)
    dst2 = jnp.concatenate([dst, pad_idx]).reshape(_EROWS, 128)
    val2 = jnp.concatenate(
        [adj_values, jnp.zeros((pad_n,), f32)]).reshape(_EROWS, 128)

    o1, o2, o3 = _sc_propagate(table0, src2, dst2, val2)

    m = _mean3(o1.reshape(-1, 128), o2.reshape(-1, 128), o3.reshape(-1, 128))
    m = m.reshape(4, _NP, _Q)
    user_all = jnp.concatenate([m[q, :_N_USERS] for q in range(4)], axis=1)
    item_all = jnp.concatenate([m[q, _N_USERS:_N] for q in range(4)], axis=1)
    return (user_all, item_all)


# speedup vs baseline: 9.3303x; 1.0040x over previous
import functools

import jax
import jax.numpy as jnp
from jax import lax
from jax.experimental import pallas as pl
from jax.experimental.pallas import tpu as pltpu
from jax.experimental.pallas import tpu_sc as plsc

_N_USERS = 50000
_N = 100000
_E = 1600000
_D = 32
_L = 3

_NS = 16
_Q = 8
_RPT = 6256
_NP = _RPT * _NS
_EP = 1638400
_EROWS = _EP // 128
_TROWS = _EROWS // _NS
_WROWS = 16
_NWIN = _TROWS // _WROWS


def _sc_propagate(table0, src2, dst2, val2):
    mesh = plsc.VectorSubcoreMesh(core_axis_name="c", subcore_axis_name="s")
    out_sds = jax.ShapeDtypeStruct((4 * _NP, _Q), jnp.float32)

    @functools.partial(
        pl.kernel,
        out_type=[out_sds] * _L,
        mesh=mesh,
        scratch_types=[
            pltpu.VMEM((_WROWS, 128), jnp.int32),
            pltpu.VMEM((_WROWS, 128), jnp.int32),
            pltpu.VMEM((_WROWS, 128), jnp.float32),
            pltpu.VMEM((_WROWS, 128, _Q), jnp.float32),
            pltpu.VMEM((_WROWS, 128), jnp.int32),
            pltpu.VMEM((_WROWS, 128), jnp.int32),
            pltpu.VMEM((_WROWS, 128), jnp.float32),
            pltpu.VMEM((_WROWS, 128, _Q), jnp.float32),
            pltpu.VMEM((128, _Q), jnp.float32),
            pltpu.VMEM_SHARED((_NP, _Q), jnp.float32),
            pltpu.SemaphoreType.DMA,
            pltpu.SemaphoreType.DMA,
            pltpu.SemaphoreType.DMA,
        ],
        compiler_params=pltpu.CompilerParams(use_tc_tiling_on_sc=False,
                                             needs_layout_passes=False),
    )
    def body(tab_hbm, src_hbm, dst_hbm, val_hbm, o1, o2, o3,
             srcA, dstA, valA, rowsA, srcB, dstB, valB, rowsB,
             zbuf, acc, lsem, gsem, ssem):
        c = lax.axis_index("c")
        s = lax.axis_index("s")
        r0 = s * _RPT
        erow0 = s * _TROWS

        iota = lax.broadcasted_iota(jnp.int32, (16,), 0)
        patv = iota >> 3
        colv = iota & 7
        lane_lo = patv == 0

        zeros16 = jnp.zeros((16,), jnp.float32)

        @pl.loop(0, 64)
        def _zfill(k):
            plsc.store_scatter(zbuf, [patv + 2 * k, colv], zeros16)

        def load_idx(w, sb, db, vb):
            wr = jnp.minimum(erow0 + w * _WROWS, _EROWS - _WROWS)
            pltpu.async_copy(src_hbm.at[pl.ds(wr, _WROWS)], sb, lsem)
            pltpu.async_copy(dst_hbm.at[pl.ds(wr, _WROWS)], db, lsem)
            pltpu.async_copy(val_hbm.at[pl.ds(wr, _WROWS)], vb, lsem)

        def drain_idx(sb, db, vb):
            pltpu.make_async_copy(src_hbm.at[pl.ds(0, _WROWS)], sb, lsem).wait()
            pltpu.make_async_copy(dst_hbm.at[pl.ds(0, _WROWS)], db, lsem).wait()
            pltpu.make_async_copy(val_hbm.at[pl.ds(0, _WROWS)], vb, lsem).wait()

        def shift(sb, qrow):
            @pl.loop(0, _WROWS)
            def _shift(j):
                for k in range(0, 128, 16):
                    sb[j, pl.ds(k, 16)] = sb[j, pl.ds(k, 16)] + qrow

        def fire_gathers(tab, sb, rb):
            for j in range(_WROWS):
                pltpu.async_copy(tab.at[sb.at[j]], rb.at[j], gsem)

        def drain_gathers(tab, sb, rb):
            for j in range(_WROWS):
                pltpu.make_async_copy(tab.at[sb.at[j]], rb.at[j], gsem).wait()

        def fire_scatters(db, rb):
            for j in range(_WROWS):
                pltpu.async_copy(rb.at[j], acc.at[db.at[j]], ssem, add=True)

        def drain_scatters(db, rb):
            for j in range(_WROWS):
                pltpu.make_async_copy(rb.at[j], acc.at[db.at[j]], ssem).wait()

        pats = [patv + 2 * t for t in range(8)]

        def scale(vb, rb):
            @pl.loop(0, _WROWS)
            def _scalej(j):
                jv = jnp.broadcast_to(j, (16,))

                for m in range(8):
                    vv = vb[j, pl.ds(m * 16, 16)]
                    basev = jnp.broadcast_to(m * 16, (16,))
                    for t in range(8):
                        pair = lax.gather(
                            vv, pats[t][:, None],
                            lax.GatherDimensionNumbers(
                                offset_dims=(), collapsed_slice_dims=(0,),
                                start_index_map=(0,)),
                            slice_sizes=(1,),
                            mode=lax.GatherScatterMode.PROMISE_IN_BOUNDS)
                        ridx = pats[t] + basev
                        rows = plsc.load_gather(rb, [jv, ridx, colv])
                        plsc.store_scatter(rb, [jv, ridx, colv], rows * pair)

        tables = [tab_hbm, o1, o2]
        outs = [o1, o2, o3]
        for l in range(_L):
            @pl.loop(0, 2)
            def _pass(qi, l=l):
                q = 2 * c + qi
                qrow = q * _NP
                tab = tables[l]

                @pl.loop(0, 48)
                def _zero(i):
                    pltpu.sync_copy(zbuf, acc.at[pl.ds(r0 + i * 128, 128)])
                pltpu.sync_copy(zbuf.at[pl.ds(0, 112)],
                                acc.at[pl.ds(r0 + 48 * 128, 112)])
                plsc.subcore_barrier()

                load_idx(0, srcA, dstA, valA)
                drain_idx(srcA, dstA, valA)
                shift(srcA, qrow)
                fire_gathers(tab, srcA, rowsA)

                @pl.loop(0, _NWIN // 2)
                def _iter(i):
                    @pl.when(i > 0)
                    def _():
                        drain_scatters(dstB, rowsB)
                    load_idx(2 * i + 1, srcB, dstB, valB)
                    drain_idx(srcB, dstB, valB)
                    shift(srcB, qrow)
                    fire_gathers(tab, srcB, rowsB)
                    drain_gathers(tab, srcA, rowsA)
                    scale(valA, rowsA)
                    fire_scatters(dstA, rowsA)

                    drain_scatters(dstA, rowsA)
                    load_idx(2 * i + 2, srcA, dstA, valA)
                    drain_idx(srcA, dstA, valA)
                    shift(srcA, qrow)
                    fire_gathers(tab, srcA, rowsA)
                    drain_gathers(tab, srcB, rowsB)
                    scale(valB, rowsB)
                    fire_scatters(dstB, rowsB)

                drain_scatters(dstB, rowsB)
                drain_gathers(tab, srcA, rowsA)

                plsc.subcore_barrier()
                pltpu.sync_copy(acc.at[pl.ds(r0, _RPT)],
                                outs[l].at[pl.ds(qrow + r0, _RPT)])

    return body(table0, src2, dst2, val2)


def _mean3(a, b, c):
    rows = a.shape[0]
    grid = 8
    bs = rows // grid

    def body(x_ref, y_ref, z_ref, o_ref):
        o_ref[...] = (x_ref[...] + y_ref[...] + z_ref[...]) * (1.0 / 3.0)

    spec = pl.BlockSpec((bs, 128), lambda i: (i, 0))
    return pl.pallas_call(
        body,
        out_shape=jax.ShapeDtypeStruct(a.shape, a.dtype),
        grid=(grid,),
        in_specs=[spec, spec, spec],
        out_specs=spec,
    )(a, b, c)


def kernel(user_emb, item_emb, adj_values, adj_indices):
    f32 = jnp.float32
    src = adj_indices[0]
    dst = adj_indices[1]
    ego = jnp.concatenate([user_emb, item_emb], axis=0)
    zpad = jnp.zeros((_NP - _N, _Q), f32)
    table0 = jnp.concatenate(
        [ego[:, 0:8], zpad, ego[:, 8:16], zpad,
         ego[:, 16:24], zpad, ego[:, 24:32], zpad], axis=0)

    pad_n = _EP - _E
    pad_idx = (jnp.arange(pad_n, dtype=jnp.int32) * 61) % _N
    src2 = jnp.concatenate([src, pad_idx]).reshape(_EROWS, 128)
    dst2 = jnp.concatenate([dst, pad_idx]).reshape(_EROWS, 128)
    val2 = jnp.concatenate(
        [adj_values, jnp.zeros((pad_n,), f32)]).reshape(_EROWS, 128)

    o1, o2, o3 = _sc_propagate(table0, src2, dst2, val2)

    m = _mean3(o1.reshape(-1, 128), o2.reshape(-1, 128), o3.reshape(-1, 128))
    m = m.reshape(4, _NP, _Q)
    user_all = jnp.concatenate([m[q, :_N_USERS] for q in range(4)], axis=1)
    item_all = jnp.concatenate([m[q, _N_USERS:_N] for q in range(4)], axis=1)
    return (user_all, item_all)


# speedup vs baseline: 9.4569x; 1.0136x over previous
import functools

import jax
import jax.numpy as jnp
from jax import lax
from jax.experimental import pallas as pl
from jax.experimental.pallas import tpu as pltpu
from jax.experimental.pallas import tpu_sc as plsc

_N_USERS = 50000
_N = 100000
_E = 1600000
_D = 32
_L = 3

_NS = 16
_Q = 8
_RPT = 6256
_NP = _RPT * _NS
_EP = 1638400
_EROWS = _EP // 128
_TROWS = _EROWS // _NS
_WROWS = 20
_NWIN = _TROWS // _WROWS


def _sc_propagate(table0, src2, dst2, val2):
    mesh = plsc.VectorSubcoreMesh(core_axis_name="c", subcore_axis_name="s")
    out_sds = jax.ShapeDtypeStruct((4 * _NP, _Q), jnp.float32)

    @functools.partial(
        pl.kernel,
        out_type=[out_sds] * _L,
        mesh=mesh,
        scratch_types=[
            pltpu.VMEM((_WROWS, 128), jnp.int32),
            pltpu.VMEM((_WROWS, 128), jnp.int32),
            pltpu.VMEM((_WROWS, 128), jnp.float32),
            pltpu.VMEM((_WROWS, 128, _Q), jnp.float32),
            pltpu.VMEM((_WROWS, 128), jnp.int32),
            pltpu.VMEM((_WROWS, 128), jnp.int32),
            pltpu.VMEM((_WROWS, 128), jnp.float32),
            pltpu.VMEM((_WROWS, 128, _Q), jnp.float32),
            pltpu.VMEM((128, _Q), jnp.float32),
            pltpu.VMEM_SHARED((_NP, _Q), jnp.float32),
            pltpu.SemaphoreType.DMA,
            pltpu.SemaphoreType.DMA,
            pltpu.SemaphoreType.DMA,
        ],
        compiler_params=pltpu.CompilerParams(use_tc_tiling_on_sc=False,
                                             needs_layout_passes=False),
    )
    def body(tab_hbm, src_hbm, dst_hbm, val_hbm, o1, o2, o3,
             srcA, dstA, valA, rowsA, srcB, dstB, valB, rowsB,
             zbuf, acc, lsem, gsem, ssem):
        c = lax.axis_index("c")
        s = lax.axis_index("s")
        r0 = s * _RPT
        erow0 = s * _TROWS

        iota = lax.broadcasted_iota(jnp.int32, (16,), 0)
        patv = iota >> 3
        colv = iota & 7
        lane_lo = patv == 0

        zeros16 = jnp.zeros((16,), jnp.float32)

        @pl.loop(0, 64)
        def _zfill(k):
            plsc.store_scatter(zbuf, [patv + 2 * k, colv], zeros16)

        def load_idx(w, sb, db, vb):
            wr = jnp.minimum(erow0 + w * _WROWS, _EROWS - _WROWS)
            pltpu.async_copy(src_hbm.at[pl.ds(wr, _WROWS)], sb, lsem)
            pltpu.async_copy(dst_hbm.at[pl.ds(wr, _WROWS)], db, lsem)
            pltpu.async_copy(val_hbm.at[pl.ds(wr, _WROWS)], vb, lsem)

        def drain_idx(sb, db, vb):
            pltpu.make_async_copy(src_hbm.at[pl.ds(0, _WROWS)], sb, lsem).wait()
            pltpu.make_async_copy(dst_hbm.at[pl.ds(0, _WROWS)], db, lsem).wait()
            pltpu.make_async_copy(val_hbm.at[pl.ds(0, _WROWS)], vb, lsem).wait()

        def shift(sb, qrow):
            @pl.loop(0, _WROWS)
            def _shift(j):
                for k in range(0, 128, 16):
                    sb[j, pl.ds(k, 16)] = sb[j, pl.ds(k, 16)] + qrow

        def fire_gathers(tab, sb, rb):
            for j in range(_WROWS):
                pltpu.async_copy(tab.at[sb.at[j]], rb.at[j], gsem)

        def drain_gathers(tab, sb, rb):
            for j in range(_WROWS):
                pltpu.make_async_copy(tab.at[sb.at[j]], rb.at[j], gsem).wait()

        def fire_scatters(db, rb):
            for j in range(_WROWS):
                pltpu.async_copy(rb.at[j], acc.at[db.at[j]], ssem, add=True)

        def drain_scatters(db, rb):
            for j in range(_WROWS):
                pltpu.make_async_copy(rb.at[j], acc.at[db.at[j]], ssem).wait()

        pats = [patv + 2 * t for t in range(8)]

        def scale(vb, rb):
            @pl.loop(0, _WROWS)
            def _scalej(j):
                jv = jnp.broadcast_to(j, (16,))

                for m in range(8):
                    vv = vb[j, pl.ds(m * 16, 16)]
                    basev = jnp.broadcast_to(m * 16, (16,))
                    for t in range(8):
                        pair = lax.gather(
                            vv, pats[t][:, None],
                            lax.GatherDimensionNumbers(
                                offset_dims=(), collapsed_slice_dims=(0,),
                                start_index_map=(0,)),
                            slice_sizes=(1,),
                            mode=lax.GatherScatterMode.PROMISE_IN_BOUNDS)
                        ridx = pats[t] + basev
                        rows = plsc.load_gather(rb, [jv, ridx, colv])
                        plsc.store_scatter(rb, [jv, ridx, colv], rows * pair)

        tables = [tab_hbm, o1, o2]
        outs = [o1, o2, o3]
        for l in range(_L):
            @pl.loop(0, 2)
            def _pass(qi, l=l):
                q = 2 * c + qi
                qrow = q * _NP
                tab = tables[l]

                @pl.loop(0, 48)
                def _zero(i):
                    pltpu.sync_copy(zbuf, acc.at[pl.ds(r0 + i * 128, 128)])
                pltpu.sync_copy(zbuf.at[pl.ds(0, 112)],
                                acc.at[pl.ds(r0 + 48 * 128, 112)])
                plsc.subcore_barrier()

                load_idx(0, srcA, dstA, valA)
                drain_idx(srcA, dstA, valA)
                shift(srcA, qrow)
                fire_gathers(tab, srcA, rowsA)

                @pl.loop(0, _NWIN // 2)
                def _iter(i):
                    @pl.when(i > 0)
                    def _():
                        drain_scatters(dstB, rowsB)
                    load_idx(2 * i + 1, srcB, dstB, valB)
                    drain_idx(srcB, dstB, valB)
                    shift(srcB, qrow)
                    fire_gathers(tab, srcB, rowsB)
                    drain_gathers(tab, srcA, rowsA)
                    scale(valA, rowsA)
                    fire_scatters(dstA, rowsA)

                    drain_scatters(dstA, rowsA)
                    load_idx(2 * i + 2, srcA, dstA, valA)
                    drain_idx(srcA, dstA, valA)
                    shift(srcA, qrow)
                    fire_gathers(tab, srcA, rowsA)
                    drain_gathers(tab, srcB, rowsB)
                    scale(valB, rowsB)
                    fire_scatters(dstB, rowsB)

                drain_scatters(dstB, rowsB)
                drain_gathers(tab, srcA, rowsA)

                plsc.subcore_barrier()
                pltpu.sync_copy(acc.at[pl.ds(r0, _RPT)],
                                outs[l].at[pl.ds(qrow + r0, _RPT)])

    return body(table0, src2, dst2, val2)


def _mean3(a, b, c):
    rows = a.shape[0]
    grid = 8
    bs = rows // grid

    def body(x_ref, y_ref, z_ref, o_ref):
        o_ref[...] = (x_ref[...] + y_ref[...] + z_ref[...]) * (1.0 / 3.0)

    spec = pl.BlockSpec((bs, 128), lambda i: (i, 0))
    return pl.pallas_call(
        body,
        out_shape=jax.ShapeDtypeStruct(a.shape, a.dtype),
        grid=(grid,),
        in_specs=[spec, spec, spec],
        out_specs=spec,
    )(a, b, c)


def kernel(user_emb, item_emb, adj_values, adj_indices):
    f32 = jnp.float32
    src = adj_indices[0]
    dst = adj_indices[1]
    ego = jnp.concatenate([user_emb, item_emb], axis=0)
    zpad = jnp.zeros((_NP - _N, _Q), f32)
    table0 = jnp.concatenate(
        [ego[:, 0:8], zpad, ego[:, 8:16], zpad,
         ego[:, 16:24], zpad, ego[:, 24:32], zpad], axis=0)

    pad_n = _EP - _E
    pad_idx = (jnp.arange(pad_n, dtype=jnp.int32) * 61) % _N
    src2 = jnp.concatenate([src, pad_idx]).reshape(_EROWS, 128)
    dst2 = jnp.concatenate([dst, pad_idx]).reshape(_EROWS, 128)
    val2 = jnp.concatenate(
        [adj_values, jnp.zeros((pad_n,), f32)]).reshape(_EROWS, 128)

    o1, o2, o3 = _sc_propagate(table0, src2, dst2, val2)

    m = _mean3(o1.reshape(-1, 128), o2.reshape(-1, 128), o3.reshape(-1, 128))
    m = m.reshape(4, _NP, _Q)
    user_all = jnp.concatenate([m[q, :_N_USERS] for q in range(4)], axis=1)
    item_all = jnp.concatenate([m[q, _N_USERS:_N] for q in range(4)], axis=1)
    return (user_all, item_all)
